# Initial kernel scaffold; baseline (speedup 1.0000x reference)
import jax
import jax.numpy as jnp
from jax.experimental import pallas as pl


def kernel(features, coords, W_feat, b_feat, Wp1, bp1, Wp2, bp2, Wq, bq, Wk, bk, Wv, bv, Wo, bo, ln_g, ln_b):
    raise NotImplementedError("write your pallas kernel here")


# speedup vs baseline: 14.7104x; 14.7104x over previous
import math

import jax
import jax.numpy as jnp
from jax.experimental import pallas as pl
from jax.experimental.pallas import tpu as pltpu

_B, _N, _DIN, _D = 4, 4096, 128, 128
_H = 4
_DH = _D // _H
_GS = (8, 8)
_IMG = (256.0, 256.0)
_NB = 1
_NC = _GS[0] * _GS[1]
_K = (2 * _NB + 1) ** 2

_BLK1 = 1024
_BLK2 = 512


def _cell_xy(co, blk):
    cw = _IMG[0] / _GS[0]
    ch = _IMG[1] / _GS[1]
    gx = jnp.clip((co[:, 0:1] / cw).astype(jnp.int32), 0, _GS[0] - 1)
    gy = jnp.clip((co[:, 1:2] / ch).astype(jnp.int32), 0, _GS[1] - 1)
    return gx, gy


def _bin_kernel(x_ref, c_ref, csum_ref, cntb_ref, cntr_ref):
    n = pl.program_id(1)
    x = x_ref[0]
    co = c_ref[0]
    gx, gy = _cell_xy(co, _BLK1)
    cell = gx * _GS[1] + gy
    lane = jax.lax.broadcasted_iota(jnp.int32, (_BLK1, _NC), 1)
    oh = (cell == lane).astype(jnp.float32)
    dn = (((0,), (0,)), ((), ()))
    ps = jax.lax.dot_general(oh, x, dn, preferred_element_type=jnp.float32)
    pc = jax.lax.dot_general(oh, jnp.ones((_BLK1, _D), jnp.float32), dn,
                             preferred_element_type=jnp.float32)
    pr = jnp.sum(oh, axis=0, keepdims=True)

    @pl.when(n == 0)
    def _():
        csum_ref[0] = ps
        cntb_ref[0] = pc
        cntr_ref[0] = pr

    @pl.when(n != 0)
    def _():
        csum_ref[0] += ps
        cntb_ref[0] += pc
        cntr_ref[0] += pr


def _attn_kernel(x_ref, c_ref, csum_ref, cntb_ref, cntr_ref, offs_ref,
                 wf_ref, bf_ref, wp1_ref, bp1_ref, wp2_ref, bp2_ref,
                 wq_ref, bq_ref, wk_ref, bk_ref, wv_ref, bv_ref,
                 wo_ref, bo_ref, lng_ref, lnb_ref, out_ref):
    f32 = jnp.float32
    feat = jnp.dot(x_ref[0], wf_ref[...], preferred_element_type=f32) + bf_ref[...]

    cnt = cntb_ref[0]
    csum_feat = (jnp.dot(csum_ref[0], wf_ref[...], preferred_element_type=f32)
                 + cnt * bf_ref[...])
    cmean = csum_feat / jnp.maximum(cnt, 1.0)
    kcell = jnp.dot(cmean, wk_ref[...], preferred_element_type=f32) + bk_ref[...]
    vcell = jnp.dot(cmean, wv_ref[...], preferred_element_type=f32) + bv_ref[...]

    pe = jnp.maximum(
        jnp.dot(offs_ref[...], wp1_ref[...], preferred_element_type=f32)
        + bp1_ref[...], 0.0)
    pe = jnp.dot(pe, wp2_ref[...], preferred_element_type=f32) + bp2_ref[...]
    pk = jnp.dot(pe, wk_ref[...], preferred_element_type=f32)
    pv = jnp.dot(pe, wv_ref[...], preferred_element_type=f32)

    q = jnp.dot(feat, wq_ref[...], preferred_element_type=f32) + bq_ref[...]

    co = c_ref[0]
    gx, gy = _cell_xy(co, _BLK2)
    lane_c = jax.lax.broadcasted_iota(jnp.int32, (_BLK2, _NC), 1)
    cx = lane_c // _GS[1]
    cy = lane_c % _GS[1]
    dx = cx - gx
    dy = cy - gy
    geo = (jnp.abs(dx) <= _NB) & (jnp.abs(dy) <= _NB)
    occ = cntr_ref[0] > 0.0
    valid = geo & occ
    jmap = (dx + _NB) * (2 * _NB + 1) + (dy + _NB)
    scale = f32(1.0 / math.sqrt(_DH))
    neg = f32(-1e9)

    lane_d = jax.lax.broadcasted_iota(jnp.int32, (_NC, _D), 1)
    lane_d16 = jax.lax.broadcasted_iota(jnp.int32, (16, _D), 1)
    dn_t = (((1,), (1,)), ((), ()))
    dn_n = (((1,), (0,)), ((), ()))
    out = jnp.zeros((_BLK2, _D), f32)
    for h in range(_H):
        mask_c = (lane_d // _DH == h).astype(f32)
        mask_p = (lane_d16 // _DH == h).astype(f32)
        s = jax.lax.dot_general(q, kcell * mask_c, dn_t,
                                preferred_element_type=f32)
        qp = jax.lax.dot_general(q, pk * mask_p, dn_t,
                                 preferred_element_type=f32)
        pos_s = jnp.zeros((_BLK2, _NC), f32)
        for j in range(_K):
            pos_s = pos_s + jnp.where(jmap == j, qp[:, j:j + 1], 0.0)
        s = (s + pos_s) * scale
        s = jnp.where(valid, s, neg)
        m = jnp.max(s, axis=1, keepdims=True)
        e = jnp.exp(s - m)
        attn = e / jnp.sum(e, axis=1, keepdims=True)
        out = out + jax.lax.dot_general(attn, vcell * mask_c, dn_n,
                                        preferred_element_type=f32)
        pvh = pv * mask_p
        for j in range(_K):
            aj = jnp.sum(jnp.where(jmap == j, attn, 0.0), axis=1, keepdims=True)
            out = out + aj * pvh[j:j + 1, :]

    o = jnp.dot(out, wo_ref[...], preferred_element_type=f32) + bo_ref[...]
    any_valid = jnp.max(valid.astype(f32), axis=1, keepdims=True) > 0.0
    enh = feat + jnp.where(any_valid, o, 0.0)
    mu = jnp.mean(enh, axis=1, keepdims=True)
    var = jnp.mean((enh - mu) ** 2, axis=1, keepdims=True)
    y = (enh - mu) / jnp.sqrt(var + 1e-5) * lng_ref[...] + lnb_ref[...]
    out_ref[0] = y


def kernel(features, coords, W_feat, b_feat, Wp1, bp1, Wp2, bp2, Wq, bq,
           Wk, bk, Wv, bv, Wo, bo, ln_g, ln_b):
    f32 = jnp.float32
    row = lambda v: v.reshape(1, -1).astype(f32)

    nb1 = _N // _BLK1
    csum, cntb, cntr = pl.pallas_call(
        _bin_kernel,
        grid=(_B, nb1),
        in_specs=[
            pl.BlockSpec((1, _BLK1, _DIN), lambda b, n: (b, n, 0)),
            pl.BlockSpec((1, _BLK1, 2), lambda b, n: (b, n, 0)),
        ],
        out_specs=[
            pl.BlockSpec((1, _NC, _DIN), lambda b, n: (b, 0, 0)),
            pl.BlockSpec((1, _NC, _D), lambda b, n: (b, 0, 0)),
            pl.BlockSpec((1, 1, _NC), lambda b, n: (b, 0, 0)),
        ],
        out_shape=[
            jax.ShapeDtypeStruct((_B, _NC, _DIN), f32),
            jax.ShapeDtypeStruct((_B, _NC, _D), f32),
            jax.ShapeDtypeStruct((_B, 1, _NC), f32),
        ],
        compiler_params=pltpu.CompilerParams(
            dimension_semantics=("parallel", "arbitrary")),
    )(features, coords)

    offs = jnp.zeros((16, 2), f32)
    offs_list = [[float(dx), float(dy)]
                 for dx in range(-_NB, _NB + 1) for dy in range(-_NB, _NB + 1)]
    offs = offs.at[:_K].set(jnp.array(offs_list, f32))

    nb2 = _N // _BLK2
    full = lambda shape: pl.BlockSpec(shape, lambda b, n: tuple(0 for _ in shape))
    out = pl.pallas_call(
        _attn_kernel,
        grid=(_B, nb2),
        in_specs=[
            pl.BlockSpec((1, _BLK2, _DIN), lambda b, n: (b, n, 0)),
            pl.BlockSpec((1, _BLK2, 2), lambda b, n: (b, n, 0)),
            pl.BlockSpec((1, _NC, _DIN), lambda b, n: (b, 0, 0)),
            pl.BlockSpec((1, _NC, _D), lambda b, n: (b, 0, 0)),
            pl.BlockSpec((1, 1, _NC), lambda b, n: (b, 0, 0)),
            full((16, 2)),
            full((_DIN, _D)), full((1, _D)),
            full((2, _D // 2)), full((1, _D // 2)),
            full((_D // 2, _D)), full((1, _D)),
            full((_D, _D)), full((1, _D)),
            full((_D, _D)), full((1, _D)),
            full((_D, _D)), full((1, _D)),
            full((_D, _D)), full((1, _D)),
            full((1, _D)), full((1, _D)),
        ],
        out_specs=pl.BlockSpec((1, _BLK2, _D), lambda b, n: (b, n, 0)),
        out_shape=jax.ShapeDtypeStruct((_B, _N, _D), f32),
        compiler_params=pltpu.CompilerParams(
            dimension_semantics=("parallel", "parallel")),
    )(features, coords, csum, cntb, cntr, offs,
      W_feat, row(b_feat), Wp1, row(bp1), Wp2, row(bp2),
      Wq, row(bq), Wk, row(bk), Wv, row(bv), Wo, row(bo),
      row(ln_g), row(ln_b))
    return out


# speedup vs baseline: 45.5610x; 3.0972x over previous
import math

import jax
import jax.numpy as jnp
from jax.experimental import pallas as pl
from jax.experimental.pallas import tpu as pltpu

_B, _N, _DIN, _D = 4, 4096, 128, 128
_H = 4
_DH = _D // _H
_GS = (8, 8)
_IMG = (256.0, 256.0)
_NB = 1
_NC = _GS[0] * _GS[1]
_K = (2 * _NB + 1) ** 2

_BLK1 = 1024
_BLK2 = 512


def _cell_xy(co, blk):
    cw = _IMG[0] / _GS[0]
    ch = _IMG[1] / _GS[1]
    gx = jnp.clip((co[:, 0:1] / cw).astype(jnp.int32), 0, _GS[0] - 1)
    gy = jnp.clip((co[:, 1:2] / ch).astype(jnp.int32), 0, _GS[1] - 1)
    return gx, gy


def _bin_kernel(x_ref, c_ref, csum_ref, cntb_ref, cntr_ref):
    n = pl.program_id(1)
    x = x_ref[0]
    co = c_ref[0]
    gx, gy = _cell_xy(co, _BLK1)
    cell = gx * _GS[1] + gy
    lane = jax.lax.broadcasted_iota(jnp.int32, (_BLK1, _NC), 1)
    oh = (cell == lane).astype(jnp.float32)
    dn = (((0,), (0,)), ((), ()))
    ps = jax.lax.dot_general(oh, x, dn, preferred_element_type=jnp.float32)
    pc = jax.lax.dot_general(oh, jnp.ones((_BLK1, _D), jnp.float32), dn,
                             preferred_element_type=jnp.float32)
    pr = jnp.sum(oh, axis=0, keepdims=True)

    @pl.when(n == 0)
    def _():
        csum_ref[0] = ps
        cntb_ref[0] = pc
        cntr_ref[0] = pr

    @pl.when(n != 0)
    def _():
        csum_ref[0] += ps
        cntb_ref[0] += pc
        cntr_ref[0] += pr


def _finish(feat, o, any_valid, lng, lnb, out_ref):
    enh = feat + jnp.where(any_valid, o, 0.0)
    mu = jnp.mean(enh, axis=1, keepdims=True)
    var = jnp.mean((enh - mu) ** 2, axis=1, keepdims=True)
    out_ref[0] = (enh - mu) / jnp.sqrt(var + 1e-5) * lng + lnb


def _attn_kernel(x_ref, c_ref, csum_ref, cntb_ref, cntr_ref, offs_ref,
                 wf_ref, bf_ref, wp1_ref, bp1_ref, wp2_ref, bp2_ref,
                 wq_ref, bq_ref, wk_ref, bk_ref, wv_ref, bv_ref,
                 wo_ref, bo_ref, lng_ref, lnb_ref, out_ref):
    f32 = jnp.float32
    i32 = jnp.int32
    feat = jnp.dot(x_ref[0], wf_ref[...], preferred_element_type=f32) + bf_ref[...]

    cnt = cntb_ref[0]
    csum_feat = (jnp.dot(csum_ref[0], wf_ref[...], preferred_element_type=f32)
                 + cnt * bf_ref[...])
    cmean = csum_feat / jnp.maximum(cnt, 1.0)
    kcell = jnp.dot(cmean, wk_ref[...], preferred_element_type=f32) + bk_ref[...]
    vcell = jnp.dot(cmean, wv_ref[...], preferred_element_type=f32) + bv_ref[...]

    pe = jnp.maximum(
        jnp.dot(offs_ref[...], wp1_ref[...], preferred_element_type=f32)
        + bp1_ref[...], 0.0)
    pe = jnp.dot(pe, wp2_ref[...], preferred_element_type=f32) + bp2_ref[...]
    pk = jnp.dot(pe, wk_ref[...], preferred_element_type=f32)
    pv = jnp.dot(pe, wv_ref[...], preferred_element_type=f32)

    q = jnp.dot(feat, wq_ref[...], preferred_element_type=f32) + bq_ref[...]

    co = c_ref[0]
    gx, gy = _cell_xy(co, _BLK2)
    cell = gx * _GS[1] + gy
    occ = cntr_ref[0] > 0.0
    scale = f32(1.0 / math.sqrt(_DH))
    neg = f32(-1e9)
    dn_t = (((1,), (1,)), ((), ()))
    dn_n = (((1,), (0,)), ((), ()))
    lng = lng_ref[...]
    lnb = lnb_ref[...]

    cmin = jnp.min(cell)
    cmax = jnp.max(cell)

    @pl.when(cmin == cmax)
    def _uniform():
        pgx = gx[0:1, 0:1]
        pgy = gy[0:1, 0:1]
        row = jax.lax.broadcasted_iota(i32, (_NC, _D), 0)
        rx = row // _GS[1]
        ry = row % _GS[1]
        dxr = rx - pgx
        dyr = ry - pgy
        geor = (jnp.abs(dxr) <= _NB) & (jnp.abs(dyr) <= _NB)
        jr = (dxr + _NB) * (2 * _NB + 1) + (dyr + _NB)
        keff = kcell
        veff = vcell
        for j in range(_K):
            selj = geor & (jr == j)
            keff = keff + jnp.where(selj, pk[j:j + 1, :], 0.0)
            veff = veff + jnp.where(selj, pv[j:j + 1, :], 0.0)

        lane_d = jax.lax.broadcasted_iota(i32, (_NC, _D), 1)
        hmask = [(lane_d // _DH == h) for h in range(_H)]
        k2a = jnp.concatenate([jnp.where(hmask[0], keff, 0.0),
                               jnp.where(hmask[1], keff, 0.0)], axis=0)
        k2b = jnp.concatenate([jnp.where(hmask[2], keff, 0.0),
                               jnp.where(hmask[3], keff, 0.0)], axis=0)
        v2a = jnp.concatenate([jnp.where(hmask[0], veff, 0.0),
                               jnp.where(hmask[1], veff, 0.0)], axis=0)
        v2b = jnp.concatenate([jnp.where(hmask[2], veff, 0.0),
                               jnp.where(hmask[3], veff, 0.0)], axis=0)

        l2 = jax.lax.broadcasted_iota(i32, (1, 2 * _NC), 1) % _NC
        cx2 = l2 // _GS[1]
        cy2 = l2 % _GS[1]
        geo2 = ((jnp.abs(cx2 - pgx) <= _NB) & (jnp.abs(cy2 - pgy) <= _NB))
        cnt2 = jnp.concatenate([cntr_ref[0], cntr_ref[0]], axis=1)
        valid2 = geo2 & (cnt2 > 0.0)

        sa = jax.lax.dot_general(q, k2a, dn_t, preferred_element_type=f32)
        sb = jax.lax.dot_general(q, k2b, dn_t, preferred_element_type=f32)
        sa = jnp.where(valid2, sa * scale, neg)
        sb = jnp.where(valid2, sb * scale, neg)
        ea = jnp.exp(sa - jnp.max(sa, axis=1, keepdims=True))
        eb = jnp.exp(sb - jnp.max(sb, axis=1, keepdims=True))
        gi = jax.lax.broadcasted_iota(i32, (2 * _NC, 2 * _NC), 0)
        gj = jax.lax.broadcasted_iota(i32, (2 * _NC, 2 * _NC), 1)
        gsum = ((gi // _NC) == (gj // _NC)).astype(f32)
        attn_a = ea / jax.lax.dot_general(ea, gsum, dn_n,
                                          preferred_element_type=f32)
        attn_b = eb / jax.lax.dot_general(eb, gsum, dn_n,
                                          preferred_element_type=f32)
        out = (jax.lax.dot_general(attn_a, v2a, dn_n, preferred_element_type=f32)
               + jax.lax.dot_general(attn_b, v2b, dn_n,
                                     preferred_element_type=f32))
        o = jnp.dot(out, wo_ref[...], preferred_element_type=f32) + bo_ref[...]
        any_valid = jnp.max(valid2.astype(f32), axis=1, keepdims=True) > 0.0
        _finish(feat, o, any_valid, lng, lnb, out_ref)

    @pl.when(cmin != cmax)
    def _general():
        lane_c = jax.lax.broadcasted_iota(i32, (_BLK2, _NC), 1)
        cx = lane_c // _GS[1]
        cy = lane_c % _GS[1]
        dx = cx - gx
        dy = cy - gy
        geo = (jnp.abs(dx) <= _NB) & (jnp.abs(dy) <= _NB)
        valid = geo & occ
        jmap = (dx + _NB) * (2 * _NB + 1) + (dy + _NB)

        lane_d = jax.lax.broadcasted_iota(i32, (_NC, _D), 1)
        lane_d16 = jax.lax.broadcasted_iota(i32, (16, _D), 1)
        out = jnp.zeros((_BLK2, _D), f32)
        for h in range(_H):
            mask_c = (lane_d // _DH == h).astype(f32)
            mask_p = (lane_d16 // _DH == h).astype(f32)
            s = jax.lax.dot_general(q, kcell * mask_c, dn_t,
                                    preferred_element_type=f32)
            qp = jax.lax.dot_general(q, pk * mask_p, dn_t,
                                     preferred_element_type=f32)
            pos_s = jnp.zeros((_BLK2, _NC), f32)
            for j in range(_K):
                pos_s = pos_s + jnp.where(jmap == j, qp[:, j:j + 1], 0.0)
            s = (s + pos_s) * scale
            s = jnp.where(valid, s, neg)
            m = jnp.max(s, axis=1, keepdims=True)
            e = jnp.exp(s - m)
            attn = e / jnp.sum(e, axis=1, keepdims=True)
            out = out + jax.lax.dot_general(attn, vcell * mask_c, dn_n,
                                            preferred_element_type=f32)
            pvh = pv * mask_p
            for j in range(_K):
                aj = jnp.sum(jnp.where(jmap == j, attn, 0.0), axis=1,
                             keepdims=True)
                out = out + aj * pvh[j:j + 1, :]

        o = jnp.dot(out, wo_ref[...], preferred_element_type=f32) + bo_ref[...]
        any_valid = jnp.max(valid.astype(f32), axis=1, keepdims=True) > 0.0
        _finish(feat, o, any_valid, lng, lnb, out_ref)


def kernel(features, coords, W_feat, b_feat, Wp1, bp1, Wp2, bp2, Wq, bq,
           Wk, bk, Wv, bv, Wo, bo, ln_g, ln_b):
    f32 = jnp.float32
    row = lambda v: v.reshape(1, -1).astype(f32)

    nb1 = _N // _BLK1
    csum, cntb, cntr = pl.pallas_call(
        _bin_kernel,
        grid=(_B, nb1),
        in_specs=[
            pl.BlockSpec((1, _BLK1, _DIN), lambda b, n: (b, n, 0)),
            pl.BlockSpec((1, _BLK1, 2), lambda b, n: (b, n, 0)),
        ],
        out_specs=[
            pl.BlockSpec((1, _NC, _DIN), lambda b, n: (b, 0, 0)),
            pl.BlockSpec((1, _NC, _D), lambda b, n: (b, 0, 0)),
            pl.BlockSpec((1, 1, _NC), lambda b, n: (b, 0, 0)),
        ],
        out_shape=[
            jax.ShapeDtypeStruct((_B, _NC, _DIN), f32),
            jax.ShapeDtypeStruct((_B, _NC, _D), f32),
            jax.ShapeDtypeStruct((_B, 1, _NC), f32),
        ],
        compiler_params=pltpu.CompilerParams(
            dimension_semantics=("parallel", "arbitrary")),
    )(features, coords)

    offs = jnp.zeros((16, 2), f32)
    offs_list = [[float(dx), float(dy)]
                 for dx in range(-_NB, _NB + 1) for dy in range(-_NB, _NB + 1)]
    offs = offs.at[:_K].set(jnp.array(offs_list, f32))

    nb2 = _N // _BLK2
    full = lambda shape: pl.BlockSpec(shape, lambda b, n: tuple(0 for _ in shape))
    out = pl.pallas_call(
        _attn_kernel,
        grid=(_B, nb2),
        in_specs=[
            pl.BlockSpec((1, _BLK2, _DIN), lambda b, n: (b, n, 0)),
            pl.BlockSpec((1, _BLK2, 2), lambda b, n: (b, n, 0)),
            pl.BlockSpec((1, _NC, _DIN), lambda b, n: (b, 0, 0)),
            pl.BlockSpec((1, _NC, _D), lambda b, n: (b, 0, 0)),
            pl.BlockSpec((1, 1, _NC), lambda b, n: (b, 0, 0)),
            full((16, 2)),
            full((_DIN, _D)), full((1, _D)),
            full((2, _D // 2)), full((1, _D // 2)),
            full((_D // 2, _D)), full((1, _D)),
            full((_D, _D)), full((1, _D)),
            full((_D, _D)), full((1, _D)),
            full((_D, _D)), full((1, _D)),
            full((_D, _D)), full((1, _D)),
            full((1, _D)), full((1, _D)),
        ],
        out_specs=pl.BlockSpec((1, _BLK2, _D), lambda b, n: (b, n, 0)),
        out_shape=jax.ShapeDtypeStruct((_B, _N, _D), f32),
        compiler_params=pltpu.CompilerParams(
            dimension_semantics=("parallel", "parallel")),
    )(features, coords, csum, cntb, cntr, offs,
      W_feat, row(b_feat), Wp1, row(bp1), Wp2, row(bp2),
      Wq, row(bq), Wk, row(bk), Wv, row(bv), Wo, row(bo),
      row(ln_g), row(ln_b))
    return out


# speedup vs baseline: 80.4327x; 1.7654x over previous
import math

import jax
import jax.numpy as jnp
from jax.experimental import pallas as pl
from jax.experimental.pallas import tpu as pltpu

_B, _N, _DIN, _D = 4, 4096, 128, 128
_H = 4
_DH = _D // _H
_GS = (8, 8)
_IMG = (256.0, 256.0)
_NB = 1
_NC = _GS[0] * _GS[1]
_K = (2 * _NB + 1) ** 2

_BLK1 = 2048
_BLK2 = 1024
_NB1 = _N // _BLK1
_NB2 = _N // _BLK2
_SCALE = 1.0 / math.sqrt(_DH)


def _cell_xy(co):
    cw = _IMG[0] / _GS[0]
    ch = _IMG[1] / _GS[1]
    gx = jnp.clip((co[:, 0:1] / cw).astype(jnp.int32), 0, _GS[0] - 1)
    gy = jnp.clip((co[:, 1:2] / ch).astype(jnp.int32), 0, _GS[1] - 1)
    return gx, gy


def _bin_kernel(x_ref, c_ref, offs_ref, wf_ref, bf_ref,
                wp1_ref, bp1_ref, wp2_ref, bp2_ref,
                wq_ref, bq_ref, wk_ref, bk_ref, wv_ref, bv_ref, wo_ref,
                csum_ref, cntb_ref, cntr_ref, kcell_ref, vcell_ref,
                pk_ref, pv_ref, a4_ref, c4_ref, vw4_ref, val_ref, meta_ref):
    f32 = jnp.float32
    i32 = jnp.int32
    n = pl.program_id(1)
    x = x_ref[0]
    co = c_ref[0]
    gx, gy = _cell_xy(co)
    cell = gx * _GS[1] + gy
    lane = jax.lax.broadcasted_iota(i32, (_BLK1, _NC), 1)
    oh = (cell == lane).astype(f32)
    dn0 = (((0,), (0,)), ((), ()))
    ps = jax.lax.dot_general(oh, x, dn0, preferred_element_type=f32)
    pc = jax.lax.dot_general(oh, jnp.ones((_BLK1, _D), f32), dn0,
                             preferred_element_type=f32)
    pr = jnp.sum(oh, axis=0, keepdims=True)

    @pl.when(n == 0)
    def _():
        csum_ref[0] = ps
        cntb_ref[0] = pc
        cntr_ref[0] = pr

    @pl.when(n != 0)
    def _():
        csum_ref[0] += ps
        cntb_ref[0] += pc
        cntr_ref[0] += pr

    @pl.when(n == _NB1 - 1)
    def _tables():
        cnt = cntb_ref[0]
        csum_feat = (jnp.dot(csum_ref[0], wf_ref[...],
                             preferred_element_type=f32) + cnt * bf_ref[...])
        cmean = csum_feat / jnp.maximum(cnt, 1.0)
        kcell = jnp.dot(cmean, wk_ref[...], preferred_element_type=f32) + bk_ref[...]
        vcell = jnp.dot(cmean, wv_ref[...], preferred_element_type=f32) + bv_ref[...]
        pe = jnp.maximum(
            jnp.dot(offs_ref[...], wp1_ref[...], preferred_element_type=f32)
            + bp1_ref[...], 0.0)
        pe = jnp.dot(pe, wp2_ref[...], preferred_element_type=f32) + bp2_ref[...]
        pk = jnp.dot(pe, wk_ref[...], preferred_element_type=f32)
        pv = jnp.dot(pe, wv_ref[...], preferred_element_type=f32)
        kcell_ref[0] = kcell
        vcell_ref[0] = vcell
        pk_ref[0] = pk
        pv_ref[0] = pv

        pgx = gx[0:1, 0:1]
        pgy = gy[0:1, 0:1]
        ri = jax.lax.broadcasted_iota(i32, (16, _NC), 0)
        ci = jax.lax.broadcasted_iota(i32, (16, _NC), 1)
        dxj = ri // (2 * _NB + 1) - _NB
        dyj = ri % (2 * _NB + 1) - _NB
        nx = pgx + dxj
        ny = pgy + dyj
        inb = ((nx >= 0) & (nx < _GS[0]) & (ny >= 0) & (ny < _GS[1])
               & (ri < _K))
        sel = (inb & (ci == nx * _GS[1] + ny)).astype(f32)
        k9 = jnp.dot(sel, kcell, preferred_element_type=f32) + pk
        v9 = jnp.dot(sel, vcell, preferred_element_type=f32) + pv
        l16 = jax.lax.broadcasted_iota(i32, (16, _D), 1)
        k4 = jnp.concatenate(
            [jnp.where(l16 // _DH == h, k9, 0.0) for h in range(_H)], axis=0)
        v4 = jnp.concatenate(
            [jnp.where(l16 // _DH == h, v9, 0.0) for h in range(_H)], axis=0)
        dn_t = (((1,), (1,)), ((), ()))
        a4_ref[0] = jax.lax.dot_general(
            wq_ref[...], k4, dn_t, preferred_element_type=f32) * _SCALE
        c4_ref[0] = jax.lax.dot_general(
            bq_ref[...], k4, dn_t, preferred_element_type=f32) * _SCALE
        vw4_ref[0] = jnp.dot(v4, wo_ref[...], preferred_element_type=f32)

        occf = (cntr_ref[0] > 0.0).astype(f32)
        sel4 = jnp.concatenate([sel, sel, sel, sel], axis=0)
        occrow = jax.lax.dot_general(occf, sel4, dn_t,
                                     preferred_element_type=f32)
        validf = (occrow > 0.0).astype(f32)
        val_ref[0] = validf
        anyv = jnp.max(validf, axis=1, keepdims=True)
        pc0f = (pgx * _GS[1] + pgy).astype(f32)
        l128 = jax.lax.broadcasted_iota(i32, (1, _D), 1)
        meta_ref[0] = (jnp.where(l128 == 0, pc0f, 0.0)
                       + jnp.where(l128 == 1, anyv, 0.0))


def _finish(feat, o, any_valid, lng, lnb, out_ref):
    enh = feat + jnp.where(any_valid, o, 0.0)
    mu = jnp.mean(enh, axis=1, keepdims=True)
    var = jnp.mean((enh - mu) ** 2, axis=1, keepdims=True)
    out_ref[0] = (enh - mu) / jnp.sqrt(var + 1e-5) * lng + lnb


def _attn_kernel(x_ref, c_ref, a4_ref, c4_ref, vw4_ref, val_ref, meta_ref,
                 kcell_ref, vcell_ref, pk_ref, pv_ref, cntr_ref,
                 wf_ref, bf_ref, wq_ref, bq_ref, wo_ref, bo_ref,
                 lng_ref, lnb_ref, out_ref):
    f32 = jnp.float32
    i32 = jnp.int32
    feat = jnp.dot(x_ref[0], wf_ref[...], preferred_element_type=f32) + bf_ref[...]
    co = c_ref[0]
    gx, gy = _cell_xy(co)
    cell = gx * _GS[1] + gy
    scale = f32(_SCALE)
    neg = f32(-1e9)
    dn_t = (((1,), (1,)), ((), ()))
    dn_n = (((1,), (0,)), ((), ()))
    lng = lng_ref[...]
    lnb = lnb_ref[...]

    cmin = jnp.min(cell)
    cmax = jnp.max(cell)
    meta = meta_ref[0]
    pc0s = jnp.min(meta[0:1, 0:1])
    uniform = (cmin == cmax) & (cmin.astype(f32) == pc0s)

    @pl.when(uniform)
    def _fast():
        s = (jax.lax.dot_general(feat, a4_ref[0], dn_n,
                                 preferred_element_type=f32)
             + c4_ref[0])
        validrow = val_ref[0] > 0.0
        s = jnp.where(validrow, s, neg)
        m = jnp.max(s, axis=1, keepdims=True)
        e = jnp.exp(s - m)
        gi = jax.lax.broadcasted_iota(i32, (4 * 16, 4 * 16), 0)
        gj = jax.lax.broadcasted_iota(i32, (4 * 16, 4 * 16), 1)
        g16 = ((gi // 16) == (gj // 16)).astype(f32)
        attn = e / jax.lax.dot_general(e, g16, dn_n,
                                       preferred_element_type=f32)
        o = (jax.lax.dot_general(attn, vw4_ref[0], dn_n,
                                 preferred_element_type=f32) + bo_ref[...])
        any_valid = meta[0:1, 1:2] > 0.0
        _finish(feat, o, any_valid, lng, lnb, out_ref)

    @pl.when(jnp.logical_not(uniform))
    def _general():
        kcell = kcell_ref[0]
        vcell = vcell_ref[0]
        pk = pk_ref[0]
        pv = pv_ref[0]
        occ = cntr_ref[0] > 0.0
        q = jnp.dot(feat, wq_ref[...], preferred_element_type=f32) + bq_ref[...]
        lane_c = jax.lax.broadcasted_iota(i32, (_BLK2, _NC), 1)
        cx = lane_c // _GS[1]
        cy = lane_c % _GS[1]
        dx = cx - gx
        dy = cy - gy
        geo = (jnp.abs(dx) <= _NB) & (jnp.abs(dy) <= _NB)
        valid = geo & occ
        jmap = (dx + _NB) * (2 * _NB + 1) + (dy + _NB)

        lane_d = jax.lax.broadcasted_iota(i32, (_NC, _D), 1)
        lane_d16 = jax.lax.broadcasted_iota(i32, (16, _D), 1)
        out = jnp.zeros((_BLK2, _D), f32)
        for h in range(_H):
            mask_c = (lane_d // _DH == h).astype(f32)
            mask_p = (lane_d16 // _DH == h).astype(f32)
            s = jax.lax.dot_general(q, kcell * mask_c, dn_t,
                                    preferred_element_type=f32)
            qp = jax.lax.dot_general(q, pk * mask_p, dn_t,
                                     preferred_element_type=f32)
            pos_s = jnp.zeros((_BLK2, _NC), f32)
            for j in range(_K):
                pos_s = pos_s + jnp.where(jmap == j, qp[:, j:j + 1], 0.0)
            s = (s + pos_s) * scale
            s = jnp.where(valid, s, neg)
            m = jnp.max(s, axis=1, keepdims=True)
            e = jnp.exp(s - m)
            attn = e / jnp.sum(e, axis=1, keepdims=True)
            out = out + jax.lax.dot_general(attn, vcell * mask_c, dn_n,
                                            preferred_element_type=f32)
            pvh = pv * mask_p
            for j in range(_K):
                aj = jnp.sum(jnp.where(jmap == j, attn, 0.0), axis=1,
                             keepdims=True)
                out = out + aj * pvh[j:j + 1, :]

        o = jnp.dot(out, wo_ref[...], preferred_element_type=f32) + bo_ref[...]
        any_valid = jnp.max(valid.astype(f32), axis=1, keepdims=True) > 0.0
        _finish(feat, o, any_valid, lng, lnb, out_ref)


def kernel(features, coords, W_feat, b_feat, Wp1, bp1, Wp2, bp2, Wq, bq,
           Wk, bk, Wv, bv, Wo, bo, ln_g, ln_b):
    f32 = jnp.float32
    row = lambda v: v.reshape(1, -1).astype(f32)
    full = lambda shape: pl.BlockSpec(shape, lambda b, n: tuple(0 for _ in shape))
    per_b = lambda s1, s2: pl.BlockSpec((1, s1, s2), lambda b, n: (b, 0, 0))

    offs = jnp.zeros((16, 2), f32)
    offs_list = [[float(dx), float(dy)]
                 for dx in range(-_NB, _NB + 1) for dy in range(-_NB, _NB + 1)]
    offs = offs.at[:_K].set(jnp.array(offs_list, f32))

    shp = jax.ShapeDtypeStruct
    tables = pl.pallas_call(
        _bin_kernel,
        grid=(_B, _NB1),
        in_specs=[
            pl.BlockSpec((1, _BLK1, _DIN), lambda b, n: (b, n, 0)),
            pl.BlockSpec((1, _BLK1, 2), lambda b, n: (b, n, 0)),
            full((16, 2)),
            full((_DIN, _D)), full((1, _D)),
            full((2, _D // 2)), full((1, _D // 2)),
            full((_D // 2, _D)), full((1, _D)),
            full((_D, _D)), full((1, _D)),
            full((_D, _D)), full((1, _D)),
            full((_D, _D)), full((1, _D)),
            full((_D, _D)),
        ],
        out_specs=[
            per_b(_NC, _DIN), per_b(_NC, _D), per_b(1, _NC),
            per_b(_NC, _D), per_b(_NC, _D), per_b(16, _D), per_b(16, _D),
            per_b(_D, 4 * 16), per_b(1, 4 * 16), per_b(4 * 16, _D),
            per_b(1, 4 * 16), per_b(1, _D),
        ],
        out_shape=[
            shp((_B, _NC, _DIN), f32), shp((_B, _NC, _D), f32),
            shp((_B, 1, _NC), f32),
            shp((_B, _NC, _D), f32), shp((_B, _NC, _D), f32),
            shp((_B, 16, _D), f32), shp((_B, 16, _D), f32),
            shp((_B, _D, 4 * 16), f32), shp((_B, 1, 4 * 16), f32),
            shp((_B, 4 * 16, _D), f32), shp((_B, 1, 4 * 16), f32),
            shp((_B, 1, _D), f32),
        ],
        compiler_params=pltpu.CompilerParams(
            dimension_semantics=("parallel", "arbitrary")),
    )(features, coords, offs, W_feat, row(b_feat), Wp1, row(bp1), Wp2,
      row(bp2), Wq, row(bq), Wk, row(bk), Wv, row(bv), Wo)
    (csum, cntb, cntr, kcell, vcell, pk, pv, a4, c4, vw4, val, meta) = tables

    out = pl.pallas_call(
        _attn_kernel,
        grid=(_B, _NB2),
        in_specs=[
            pl.BlockSpec((1, _BLK2, _DIN), lambda b, n: (b, n, 0)),
            pl.BlockSpec((1, _BLK2, 2), lambda b, n: (b, n, 0)),
            per_b(_D, 4 * 16), per_b(1, 4 * 16), per_b(4 * 16, _D),
            per_b(1, 4 * 16), per_b(1, _D),
            per_b(_NC, _D), per_b(_NC, _D), per_b(16, _D), per_b(16, _D),
            per_b(1, _NC),
            full((_DIN, _D)), full((1, _D)),
            full((_D, _D)), full((1, _D)),
            full((_D, _D)), full((1, _D)),
            full((1, _D)), full((1, _D)),
        ],
        out_specs=pl.BlockSpec((1, _BLK2, _D), lambda b, n: (b, n, 0)),
        out_shape=shp((_B, _N, _D), f32),
        compiler_params=pltpu.CompilerParams(
            dimension_semantics=("parallel", "parallel")),
    )(features, coords, a4, c4, vw4, val, meta, kcell, vcell, pk, pv, cntr,
      W_feat, row(b_feat), Wq, row(bq), Wo, row(bo), row(ln_g), row(ln_b))
    return out


# speedup vs baseline: 103.8928x; 1.2917x over previous
import math

import jax
import jax.numpy as jnp
from jax.experimental import pallas as pl
from jax.experimental.pallas import tpu as pltpu

_B, _N, _DIN, _D = 4, 4096, 128, 128
_H = 4
_DH = _D // _H
_GS = (8, 8)
_IMG = (256.0, 256.0)
_NB = 1
_NC = _GS[0] * _GS[1]
_K = (2 * _NB + 1) ** 2

_BLK = 2048
_NBLK = _N // _BLK
_SCALE = 1.0 / math.sqrt(_DH)


def _cell_xy(co):
    cw = _IMG[0] / _GS[0]
    ch = _IMG[1] / _GS[1]
    gx = jnp.clip((co[:, 0:1] / cw).astype(jnp.int32), 0, _GS[0] - 1)
    gy = jnp.clip((co[:, 1:2] / ch).astype(jnp.int32), 0, _GS[1] - 1)
    return gx, gy


def _finish(feat, o, any_valid, lng, lnb, out_ref):
    enh = feat + jnp.where(any_valid, o, 0.0)
    mu = jnp.mean(enh, axis=1, keepdims=True)
    var = jnp.mean((enh - mu) ** 2, axis=1, keepdims=True)
    out_ref[0] = (enh - mu) / jnp.sqrt(var + 1e-5) * lng + lnb


def _fused_kernel(x_ref, c_ref, offs_ref, wf_ref, bf_ref,
                  wp1_ref, bp1_ref, wp2_ref, bp2_ref,
                  wq_ref, bq_ref, wk_ref, bk_ref, wv_ref, bv_ref,
                  wo_ref, bo_ref, lng_ref, lnb_ref, out_ref,
                  kcell_ref, vcell_ref, pk_ref, pv_ref, occ_ref,
                  a4_ref, c4_ref, vw4_ref, val_ref, meta_ref):
    f32 = jnp.float32
    i32 = jnp.int32
    n = pl.program_id(1)
    dn_t = (((1,), (1,)), ((), ()))
    dn_n = (((1,), (0,)), ((), ()))

    @pl.when(n == 0)
    def _tables():
        x = x_ref[0]
        co = c_ref[0]
        gx, gy = _cell_xy(co)
        cell = gx * _GS[1] + gy
        lane = jax.lax.broadcasted_iota(i32, (_N, _NC), 1)
        oh = (cell == lane).astype(f32)
        dn0 = (((0,), (0,)), ((), ()))
        csum = jax.lax.dot_general(oh, x, dn0, preferred_element_type=f32)
        cntr = jnp.sum(oh, axis=0, keepdims=True)
        cntc = jnp.transpose(cntr)
        occ_ref[...] = cntr

        csum_feat = (jnp.dot(csum, wf_ref[...], preferred_element_type=f32)
                     + cntc * bf_ref[...])
        cmean = csum_feat / jnp.maximum(cntc, 1.0)
        kcell = (jnp.dot(cmean, wk_ref[...], preferred_element_type=f32)
                 + bk_ref[...])
        vcell = (jnp.dot(cmean, wv_ref[...], preferred_element_type=f32)
                 + bv_ref[...])
        pe = jnp.maximum(
            jnp.dot(offs_ref[...], wp1_ref[...], preferred_element_type=f32)
            + bp1_ref[...], 0.0)
        pe = jnp.dot(pe, wp2_ref[...], preferred_element_type=f32) + bp2_ref[...]
        pk = jnp.dot(pe, wk_ref[...], preferred_element_type=f32)
        pv = jnp.dot(pe, wv_ref[...], preferred_element_type=f32)
        kcell_ref[...] = kcell
        vcell_ref[...] = vcell
        pk_ref[...] = pk
        pv_ref[...] = pv

        pgx = gx[0:1, 0:1]
        pgy = gy[0:1, 0:1]
        ri = jax.lax.broadcasted_iota(i32, (16, _NC), 0)
        ci = jax.lax.broadcasted_iota(i32, (16, _NC), 1)
        dxj = ri // (2 * _NB + 1) - _NB
        dyj = ri % (2 * _NB + 1) - _NB
        nx = pgx + dxj
        ny = pgy + dyj
        inb = ((nx >= 0) & (nx < _GS[0]) & (ny >= 0) & (ny < _GS[1])
               & (ri < _K))
        sel = (inb & (ci == nx * _GS[1] + ny)).astype(f32)
        k9 = jnp.dot(sel, kcell, preferred_element_type=f32) + pk
        v9 = jnp.dot(sel, vcell, preferred_element_type=f32) + pv
        l16 = jax.lax.broadcasted_iota(i32, (16, _D), 1)
        k4 = jnp.concatenate(
            [jnp.where(l16 // _DH == h, k9, 0.0) for h in range(_H)], axis=0)
        v4 = jnp.concatenate(
            [jnp.where(l16 // _DH == h, v9, 0.0) for h in range(_H)], axis=0)
        a4_ref[...] = jax.lax.dot_general(
            wq_ref[...], k4, dn_t, preferred_element_type=f32) * _SCALE
        c4_ref[...] = jax.lax.dot_general(
            bq_ref[...], k4, dn_t, preferred_element_type=f32) * _SCALE
        vw4_ref[...] = jnp.dot(v4, wo_ref[...], preferred_element_type=f32)

        occf = (cntr > 0.0).astype(f32)
        sel4 = jnp.concatenate([sel, sel, sel, sel], axis=0)
        occrow = jax.lax.dot_general(occf, sel4, dn_t,
                                     preferred_element_type=f32)
        validf = (occrow > 0.0).astype(f32)
        val_ref[...] = validf
        anyv = jnp.max(validf, axis=1, keepdims=True)
        pc0f = (pgx * _GS[1] + pgy).astype(f32)
        l128 = jax.lax.broadcasted_iota(i32, (1, _D), 1)
        meta_ref[...] = (jnp.where(l128 == 0, pc0f, 0.0)
                         + jnp.where(l128 == 1, anyv, 0.0))

    @pl.when(n > 0)
    def _attend():
        start = (n - 1) * _BLK
        x = x_ref[0, pl.ds(start, _BLK), :]
        co = c_ref[0, pl.ds(start, _BLK), :]
        feat = jnp.dot(x, wf_ref[...], preferred_element_type=f32) + bf_ref[...]
        gx, gy = _cell_xy(co)
        cell = gx * _GS[1] + gy
        lng = lng_ref[...]
        lnb = lnb_ref[...]
        neg = f32(-1e9)

        cmin = jnp.min(cell)
        cmax = jnp.max(cell)
        meta = meta_ref[...]
        pc0s = jnp.min(meta[0:1, 0:1])
        uniform = (cmin == cmax) & (cmin.astype(f32) == pc0s)

        @pl.when(uniform)
        def _fast():
            s = (jax.lax.dot_general(feat, a4_ref[...], dn_n,
                                     preferred_element_type=f32)
                 + c4_ref[...])
            validrow = val_ref[...] > 0.0
            s = jnp.where(validrow, s, neg)
            m = jnp.max(s, axis=1, keepdims=True)
            e = jnp.exp(s - m)
            gi = jax.lax.broadcasted_iota(i32, (4 * 16, 4 * 16), 0)
            gj = jax.lax.broadcasted_iota(i32, (4 * 16, 4 * 16), 1)
            g16 = ((gi // 16) == (gj // 16)).astype(f32)
            attn = e / jax.lax.dot_general(e, g16, dn_n,
                                           preferred_element_type=f32)
            o = (jax.lax.dot_general(attn, vw4_ref[...], dn_n,
                                     preferred_element_type=f32) + bo_ref[...])
            any_valid = meta[0:1, 1:2] > 0.0
            _finish(feat, o, any_valid, lng, lnb, out_ref)

        @pl.when(jnp.logical_not(uniform))
        def _general():
            kcell = kcell_ref[...]
            vcell = vcell_ref[...]
            pk = pk_ref[...]
            pv = pv_ref[...]
            occ = occ_ref[...] > 0.0
            q = (jnp.dot(feat, wq_ref[...], preferred_element_type=f32)
                 + bq_ref[...])
            lane_c = jax.lax.broadcasted_iota(i32, (_BLK, _NC), 1)
            cx = lane_c // _GS[1]
            cy = lane_c % _GS[1]
            dx = cx - gx
            dy = cy - gy
            geo = (jnp.abs(dx) <= _NB) & (jnp.abs(dy) <= _NB)
            valid = geo & occ
            jmap = (dx + _NB) * (2 * _NB + 1) + (dy + _NB)
            scale = f32(_SCALE)

            lane_d = jax.lax.broadcasted_iota(i32, (_NC, _D), 1)
            lane_d16 = jax.lax.broadcasted_iota(i32, (16, _D), 1)
            out = jnp.zeros((_BLK, _D), f32)
            for h in range(_H):
                mask_c = (lane_d // _DH == h).astype(f32)
                mask_p = (lane_d16 // _DH == h).astype(f32)
                s = jax.lax.dot_general(q, kcell * mask_c, dn_t,
                                        preferred_element_type=f32)
                qp = jax.lax.dot_general(q, pk * mask_p, dn_t,
                                         preferred_element_type=f32)
                pos_s = jnp.zeros((_BLK, _NC), f32)
                for j in range(_K):
                    pos_s = pos_s + jnp.where(jmap == j, qp[:, j:j + 1], 0.0)
                s = (s + pos_s) * scale
                s = jnp.where(valid, s, neg)
                m = jnp.max(s, axis=1, keepdims=True)
                e = jnp.exp(s - m)
                attn = e / jnp.sum(e, axis=1, keepdims=True)
                out = out + jax.lax.dot_general(attn, vcell * mask_c, dn_n,
                                                preferred_element_type=f32)
                pvh = pv * mask_p
                for j in range(_K):
                    aj = jnp.sum(jnp.where(jmap == j, attn, 0.0), axis=1,
                                 keepdims=True)
                    out = out + aj * pvh[j:j + 1, :]

            o = (jnp.dot(out, wo_ref[...], preferred_element_type=f32)
                 + bo_ref[...])
            any_valid = jnp.max(valid.astype(f32), axis=1, keepdims=True) > 0.0
            _finish(feat, o, any_valid, lng, lnb, out_ref)


def kernel(features, coords, W_feat, b_feat, Wp1, bp1, Wp2, bp2, Wq, bq,
           Wk, bk, Wv, bv, Wo, bo, ln_g, ln_b):
    f32 = jnp.float32
    row = lambda v: v.reshape(1, -1).astype(f32)
    full = lambda shape: pl.BlockSpec(shape, lambda b, n: tuple(0 for _ in shape))

    offs = jnp.zeros((16, 2), f32)
    offs_list = [[float(dx), float(dy)]
                 for dx in range(-_NB, _NB + 1) for dy in range(-_NB, _NB + 1)]
    offs = offs.at[:_K].set(jnp.array(offs_list, f32))

    out = pl.pallas_call(
        _fused_kernel,
        grid=(_B, 1 + _NBLK),
        in_specs=[
            pl.BlockSpec((1, _N, _DIN), lambda b, n: (b, 0, 0)),
            pl.BlockSpec((1, _N, 2), lambda b, n: (b, 0, 0)),
            full((16, 2)),
            full((_DIN, _D)), full((1, _D)),
            full((2, _D // 2)), full((1, _D // 2)),
            full((_D // 2, _D)), full((1, _D)),
            full((_D, _D)), full((1, _D)),
            full((_D, _D)), full((1, _D)),
            full((_D, _D)), full((1, _D)),
            full((_D, _D)), full((1, _D)),
            full((1, _D)), full((1, _D)),
        ],
        out_specs=pl.BlockSpec(
            (1, _BLK, _D), lambda b, n: (b, jnp.maximum(n - 1, 0), 0)),
        out_shape=jax.ShapeDtypeStruct((_B, _N, _D), f32),
        scratch_shapes=[
            pltpu.VMEM((_NC, _D), f32), pltpu.VMEM((_NC, _D), f32),
            pltpu.VMEM((16, _D), f32), pltpu.VMEM((16, _D), f32),
            pltpu.VMEM((1, _NC), f32),
            pltpu.VMEM((_D, 4 * 16), f32), pltpu.VMEM((1, 4 * 16), f32),
            pltpu.VMEM((4 * 16, _D), f32), pltpu.VMEM((1, 4 * 16), f32),
            pltpu.VMEM((1, _D), f32),
        ],
        compiler_params=pltpu.CompilerParams(
            dimension_semantics=("parallel", "arbitrary")),
    )(features, coords, offs, W_feat, row(b_feat), Wp1, row(bp1), Wp2,
      row(bp2), Wq, row(bq), Wk, row(bk), Wv, row(bv), Wo, row(bo),
      row(ln_g), row(ln_b))
    return out


# speedup vs baseline: 109.5790x; 1.0547x over previous
import math

import jax
import jax.numpy as jnp
from jax.experimental import pallas as pl
from jax.experimental.pallas import tpu as pltpu

_B, _N, _DIN, _D = 4, 4096, 128, 128
_H = 4
_DH = _D // _H
_GS = (8, 8)
_IMG = (256.0, 256.0)
_NB = 1
_NC = _GS[0] * _GS[1]
_K = (2 * _NB + 1) ** 2

_BLK = 2048
_NBLK = _N // _BLK
_SCALE = 1.0 / math.sqrt(_DH)


def _cell_xy(co):
    cw = _IMG[0] / _GS[0]
    ch = _IMG[1] / _GS[1]
    gx = jnp.clip((co[:, 0:1] / cw).astype(jnp.int32), 0, _GS[0] - 1)
    gy = jnp.clip((co[:, 1:2] / ch).astype(jnp.int32), 0, _GS[1] - 1)
    return gx, gy


def _finish(feat, o, any_valid, lng, lnb, out_ref):
    enh = feat + jnp.where(any_valid, o, 0.0)
    mu = jnp.mean(enh, axis=1, keepdims=True)
    var = jnp.mean((enh - mu) ** 2, axis=1, keepdims=True)
    out_ref[0] = (enh - mu) / jnp.sqrt(var + 1e-5) * lng + lnb


def _fused_kernel(x_ref, c_ref, offs_ref, wf_ref, bf_ref,
                  wp1_ref, bp1_ref, wp2_ref, bp2_ref,
                  wq_ref, bq_ref, wk_ref, bk_ref, wv_ref, bv_ref,
                  wo_ref, bo_ref, lng_ref, lnb_ref, out_ref,
                  kcell_ref, vcell_ref, pk_ref, pv_ref, occ_ref,
                  a4_ref, c4_ref, vw4_ref, val_ref, meta_ref):
    f32 = jnp.float32
    i32 = jnp.int32
    n = pl.program_id(1)
    dn_t = (((1,), (1,)), ((), ()))
    dn_n = (((1,), (0,)), ((), ()))

    @pl.when(n == 0)
    def _tables():
        x = x_ref[0]
        co = c_ref[0]
        gx, gy = _cell_xy(co)
        cell = gx * _GS[1] + gy
        lane = jax.lax.broadcasted_iota(i32, (_N, _NC), 1)
        oh = (cell == lane).astype(f32)
        dn0 = (((0,), (0,)), ((), ()))
        csum = jax.lax.dot_general(oh, x, dn0, preferred_element_type=f32)
        cntr = jnp.sum(oh, axis=0, keepdims=True)
        cntc = jnp.transpose(cntr)
        occ_ref[...] = cntr

        csum_feat = (jnp.dot(csum, wf_ref[...], preferred_element_type=f32)
                     + cntc * bf_ref[...])
        cmean = csum_feat / jnp.maximum(cntc, 1.0)
        kcell = (jnp.dot(cmean, wk_ref[...], preferred_element_type=f32)
                 + bk_ref[...])
        vcell = (jnp.dot(cmean, wv_ref[...], preferred_element_type=f32)
                 + bv_ref[...])
        pe = jnp.maximum(
            jnp.dot(offs_ref[...], wp1_ref[...], preferred_element_type=f32)
            + bp1_ref[...], 0.0)
        pe = jnp.dot(pe, wp2_ref[...], preferred_element_type=f32) + bp2_ref[...]
        pk = jnp.dot(pe, wk_ref[...], preferred_element_type=f32)
        pv = jnp.dot(pe, wv_ref[...], preferred_element_type=f32)
        kcell_ref[...] = kcell
        vcell_ref[...] = vcell
        pk_ref[...] = pk
        pv_ref[...] = pv

        pgx = gx[0:1, 0:1]
        pgy = gy[0:1, 0:1]
        ri = jax.lax.broadcasted_iota(i32, (16, _NC), 0)
        ci = jax.lax.broadcasted_iota(i32, (16, _NC), 1)
        dxj = ri // (2 * _NB + 1) - _NB
        dyj = ri % (2 * _NB + 1) - _NB
        nx = pgx + dxj
        ny = pgy + dyj
        inb = ((nx >= 0) & (nx < _GS[0]) & (ny >= 0) & (ny < _GS[1])
               & (ri < _K))
        sel = (inb & (ci == nx * _GS[1] + ny)).astype(f32)
        k9 = jnp.dot(sel, kcell, preferred_element_type=f32) + pk
        v9 = jnp.dot(sel, vcell, preferred_element_type=f32) + pv
        l16 = jax.lax.broadcasted_iota(i32, (16, _D), 1)
        k4 = jnp.concatenate(
            [jnp.where(l16 // _DH == h, k9, 0.0) for h in range(_H)], axis=0)
        v4 = jnp.concatenate(
            [jnp.where(l16 // _DH == h, v9, 0.0) for h in range(_H)], axis=0)
        a4_ref[...] = jax.lax.dot_general(
            wq_ref[...], k4, dn_t, preferred_element_type=f32) * _SCALE
        c4_ref[...] = jax.lax.dot_general(
            bq_ref[...], k4, dn_t, preferred_element_type=f32) * _SCALE
        vw4_ref[...] = jnp.dot(v4, wo_ref[...], preferred_element_type=f32)

        occf = (cntr > 0.0).astype(f32)
        sel4 = jnp.concatenate([sel, sel, sel, sel], axis=0)
        occrow = jax.lax.dot_general(occf, sel4, dn_t,
                                     preferred_element_type=f32)
        validf = (occrow > 0.0).astype(f32)
        val_ref[...] = validf
        anyv = jnp.max(validf, axis=1, keepdims=True)
        unif = (jnp.max(cntr, axis=1, keepdims=True) == f32(_N)).astype(f32)
        l128 = jax.lax.broadcasted_iota(i32, (1, _D), 1)
        meta_ref[...] = (jnp.where(l128 == 0, unif, 0.0)
                         + jnp.where(l128 == 1, anyv, 0.0))

    @pl.when(n > 0)
    def _attend():
        start = (n - 1) * _BLK
        x = x_ref[0, pl.ds(start, _BLK), :]
        feat = jnp.dot(x, wf_ref[...], preferred_element_type=f32) + bf_ref[...]
        lng = lng_ref[...]
        lnb = lnb_ref[...]
        neg = f32(-1e9)

        meta = meta_ref[...]
        uniform = jnp.min(meta[0:1, 0:1]) > 0.0

        @pl.when(uniform)
        def _fast():
            s = (jax.lax.dot_general(feat, a4_ref[...], dn_n,
                                     preferred_element_type=f32)
                 + c4_ref[...])
            validrow = val_ref[...] > 0.0
            s = jnp.where(validrow, s, neg)
            m = jnp.max(s, axis=1, keepdims=True)
            e = jnp.exp(s - m)
            gi = jax.lax.broadcasted_iota(i32, (4 * 16, 4 * 16), 0)
            gj = jax.lax.broadcasted_iota(i32, (4 * 16, 4 * 16), 1)
            g16 = ((gi // 16) == (gj // 16)).astype(f32)
            attn = e / jax.lax.dot_general(e, g16, dn_n,
                                           preferred_element_type=f32)
            o = (jax.lax.dot_general(attn, vw4_ref[...], dn_n,
                                     preferred_element_type=f32) + bo_ref[...])
            any_valid = meta[0:1, 1:2] > 0.0
            _finish(feat, o, any_valid, lng, lnb, out_ref)

        @pl.when(jnp.logical_not(uniform))
        def _general():
            kcell = kcell_ref[...]
            vcell = vcell_ref[...]
            pk = pk_ref[...]
            pv = pv_ref[...]
            occ = occ_ref[...] > 0.0
            co = c_ref[0, pl.ds(start, _BLK), :]
            gx, gy = _cell_xy(co)
            q = (jnp.dot(feat, wq_ref[...], preferred_element_type=f32)
                 + bq_ref[...])
            lane_c = jax.lax.broadcasted_iota(i32, (_BLK, _NC), 1)
            cx = lane_c // _GS[1]
            cy = lane_c % _GS[1]
            dx = cx - gx
            dy = cy - gy
            geo = (jnp.abs(dx) <= _NB) & (jnp.abs(dy) <= _NB)
            valid = geo & occ
            jmap = (dx + _NB) * (2 * _NB + 1) + (dy + _NB)
            scale = f32(_SCALE)

            lane_d = jax.lax.broadcasted_iota(i32, (_NC, _D), 1)
            lane_d16 = jax.lax.broadcasted_iota(i32, (16, _D), 1)
            out = jnp.zeros((_BLK, _D), f32)
            for h in range(_H):
                mask_c = (lane_d // _DH == h).astype(f32)
                mask_p = (lane_d16 // _DH == h).astype(f32)
                s = jax.lax.dot_general(q, kcell * mask_c, dn_t,
                                        preferred_element_type=f32)
                qp = jax.lax.dot_general(q, pk * mask_p, dn_t,
                                         preferred_element_type=f32)
                pos_s = jnp.zeros((_BLK, _NC), f32)
                for j in range(_K):
                    pos_s = pos_s + jnp.where(jmap == j, qp[:, j:j + 1], 0.0)
                s = (s + pos_s) * scale
                s = jnp.where(valid, s, neg)
                m = jnp.max(s, axis=1, keepdims=True)
                e = jnp.exp(s - m)
                attn = e / jnp.sum(e, axis=1, keepdims=True)
                out = out + jax.lax.dot_general(attn, vcell * mask_c, dn_n,
                                                preferred_element_type=f32)
                pvh = pv * mask_p
                for j in range(_K):
                    aj = jnp.sum(jnp.where(jmap == j, attn, 0.0), axis=1,
                                 keepdims=True)
                    out = out + aj * pvh[j:j + 1, :]

            o = (jnp.dot(out, wo_ref[...], preferred_element_type=f32)
                 + bo_ref[...])
            any_valid = jnp.max(valid.astype(f32), axis=1, keepdims=True) > 0.0
            _finish(feat, o, any_valid, lng, lnb, out_ref)


def kernel(features, coords, W_feat, b_feat, Wp1, bp1, Wp2, bp2, Wq, bq,
           Wk, bk, Wv, bv, Wo, bo, ln_g, ln_b):
    f32 = jnp.float32
    row = lambda v: v.reshape(1, -1).astype(f32)
    full = lambda shape: pl.BlockSpec(shape, lambda b, n: tuple(0 for _ in shape))

    offs = jnp.zeros((16, 2), f32)
    offs_list = [[float(dx), float(dy)]
                 for dx in range(-_NB, _NB + 1) for dy in range(-_NB, _NB + 1)]
    offs = offs.at[:_K].set(jnp.array(offs_list, f32))

    out = pl.pallas_call(
        _fused_kernel,
        grid=(_B, 1 + _NBLK),
        in_specs=[
            pl.BlockSpec((1, _N, _DIN), lambda b, n: (b, 0, 0)),
            pl.BlockSpec((1, _N, 2), lambda b, n: (b, 0, 0)),
            full((16, 2)),
            full((_DIN, _D)), full((1, _D)),
            full((2, _D // 2)), full((1, _D // 2)),
            full((_D // 2, _D)), full((1, _D)),
            full((_D, _D)), full((1, _D)),
            full((_D, _D)), full((1, _D)),
            full((_D, _D)), full((1, _D)),
            full((_D, _D)), full((1, _D)),
            full((1, _D)), full((1, _D)),
        ],
        out_specs=pl.BlockSpec(
            (1, _BLK, _D), lambda b, n: (b, jnp.maximum(n - 1, 0), 0)),
        out_shape=jax.ShapeDtypeStruct((_B, _N, _D), f32),
        scratch_shapes=[
            pltpu.VMEM((_NC, _D), f32), pltpu.VMEM((_NC, _D), f32),
            pltpu.VMEM((16, _D), f32), pltpu.VMEM((16, _D), f32),
            pltpu.VMEM((1, _NC), f32),
            pltpu.VMEM((_D, 4 * 16), f32), pltpu.VMEM((1, 4 * 16), f32),
            pltpu.VMEM((4 * 16, _D), f32), pltpu.VMEM((1, 4 * 16), f32),
            pltpu.VMEM((1, _D), f32),
        ],
        compiler_params=pltpu.CompilerParams(
            dimension_semantics=("parallel", "arbitrary")),
    )(features, coords, offs, W_feat, row(b_feat), Wp1, row(bp1), Wp2,
      row(bp2), Wq, row(bq), Wk, row(bk), Wv, row(bv), Wo, row(bo),
      row(ln_g), row(ln_b))
    return out
